# baseline (speedup 1.0000x reference)
import functools

import jax
import jax.numpy as jnp
from jax import lax
from jax.experimental import pallas as pl
from jax.experimental.pallas import tpu as pltpu
from jax.experimental.pallas import tpu_sc as plsc

_FEAT = 4
_NLAB = 8
_NB = 8
_H = 224
_W = 224
_ROWS = _H // 4
_L = 16
_CPR = _W // _L

_DELTA_V = 0.5
_DELTA_D = 1.5
_LOSS_WEIGHT = 0.25

_SUBS = ((0, 16), (16, 40))

_DMA_SEMS = len(_SUBS)


def _rsqrt_nr(x, iters=2):
    i = lax.bitcast_convert_type(x, jnp.int32)
    y = lax.bitcast_convert_type(
        jnp.int32(0x5F3759DF) - lax.shift_right_logical(i, 1), jnp.float32
    )
    for _ in range(iters):
        y = y * (1.5 - 0.5 * x * y * y)
    return y


def _sqrt_nr(x):
    return x * _rsqrt_nr(x, iters=3)


_mesh = plsc.VectorSubcoreMesh(
    core_axis_name="c", subcore_axis_name="s", num_cores=2, num_subcores=16
)


@functools.partial(
    pl.kernel,
    out_type=jax.ShapeDtypeStruct((32, _L), jnp.float32),
    mesh=_mesh,
    scratch_types=[
        pltpu.VMEM((_FEAT, _ROWS, _W), jnp.float32),
        pltpu.VMEM((_ROWS, _W), jnp.int32),
        pltpu.VMEM((_ROWS, _W), jnp.int32),
        pltpu.VMEM((640,), jnp.float32),
        pltpu.VMEM((128,), jnp.float32),
        pltpu.VMEM((640,), jnp.float32),
        pltpu.VMEM((2 * _L,), jnp.float32),
        pltpu.VMEM((_L,), jnp.float32),
        pltpu.VMEM_SHARED((16, 640), jnp.float32),
    ] + [pltpu.SemaphoreType.DMA] * _DMA_SEMS,
    compiler_params=pltpu.CompilerParams(needs_layout_passes=False),
)
def _emb_loss_sc(emb_hbm, gt_hbm, mask_hbm, out_hbm,
                 ev, gtv, mkv, acc, htab, tot, mutab, obuf, shacc, *sems):
    c = lax.axis_index("c")
    s = lax.axis_index("s")
    b = c * 4 + s // 4
    k = s % 4

    r0 = k * _ROWS
    copies = []
    for j, (rs, rn) in enumerate(_SUBS):
        cps = [
            pltpu.async_copy(gt_hbm.at[b, pl.ds(r0 + rs, rn), :],
                             gtv.at[pl.ds(rs, rn), :], sems[j]),
            pltpu.async_copy(mask_hbm.at[b, pl.ds(r0 + rs, rn), :],
                             mkv.at[pl.ds(rs, rn), :], sems[j]),
        ]
        for f in range(_FEAT):
            cps.append(pltpu.async_copy(
                emb_hbm.at[b, f, pl.ds(r0 + rs, rn), :],
                ev.at[f, pl.ds(rs, rn), :], sems[j]))
        copies.append(cps)

    lanes = lax.iota(jnp.int32, _L)
    zero = jnp.zeros((_L,), jnp.float32)
    ones = jnp.ones((_L,), jnp.float32)

    for g in range(640 // _L):
        acc[pl.ds(g * _L, _L)] = zero
    for g in range(128 // _L):
        htab[pl.ds(g * _L, _L)] = zero

    for j, (rs, rn) in enumerate(_SUBS):
        for cp in copies[j]:
            cp.wait()

        @plsc.parallel_loop(rs, rs + rn, step=1)
        def _p1(r):
            for ci in range(_CPR):
                gt_v = gtv[r, pl.ds(ci * _L, _L)]
                mk = mkv[r, pl.ds(ci * _L, _L)]
                sel = mk > 0
                idx = lax.shift_left(gt_v, 4) + lanes
                for f in range(_FEAT):
                    e = ev[f, r, pl.ds(ci * _L, _L)]
                    plsc.addupdate_scatter(acc, [idx + f * 128], e, mask=sel)
                plsc.addupdate_scatter(acc, [idx + 4 * 128], ones, mask=sel)
                gtv[r, pl.ds(ci * _L, _L)] = gt_v | lax.shift_left(mk, 3)

    pltpu.sync_copy(acc, shacc.at[s])
    plsc.subcore_barrier()
    row0 = (s // 4) * 4
    pltpu.sync_copy(shacc.at[row0], tot)
    for r in range(1, 4):
        pltpu.sync_copy(shacc.at[row0 + r], acc)
        for g in range(640 // _L):
            tot[pl.ds(g * _L, _L)] = tot[pl.ds(g * _L, _L)] + acc[pl.ds(g * _L, _L)]

    lab = lanes & 7
    half = lax.shift_right_logical(lanes, 3)
    base01 = lab * _L + half * 128
    s01 = zero
    s23 = zero
    scv = zero
    for t in range(_L):
        s01 = s01 + plsc.load_gather(tot, [base01 + t])
        s23 = s23 + plsc.load_gather(tot, [base01 + (256 + t)])
        scv = scv + plsc.load_gather(tot, [lab * _L + (512 + t)])

    present = scv > 0.0
    presf = jnp.where(present, 1.0, 0.0)
    cntf = jnp.where(present, scv, 1.0)
    mu01 = s01 / cntf
    mu23 = s23 / cntf
    mutab[pl.ds(0, _L)] = mu01
    mutab[pl.ds(_L, _L)] = mu23

    lane_lt8 = lanes < 8
    nf = jnp.sum(jnp.where(lane_lt8, presf, 0.0))
    validf = jnp.minimum(jnp.maximum(nf - 1.0, 0.0), 1.0)
    n_safe = jnp.maximum(nf, 1.0)
    w_vec = (presf * validf) / (cntf * n_safe)

    @plsc.parallel_loop(0, _ROWS, step=1, unroll=2)
    def _p2(r):
        for ci in range(_CPR):
            gm = gtv[r, pl.ds(ci * _L, _L)]
            gt_v = gm & 7
            sel = gm > 7
            d2 = jnp.full((_L,), 1e-12, jnp.float32)
            for f in range(_FEAT):
                e = ev[f, r, pl.ds(ci * _L, _L)]
                mu = plsc.load_gather(mutab, [gt_v + f * _NLAB])
                df = e - mu
                d2 = d2 + df * df
            dist = d2 * _rsqrt_nr(d2, iters=1)
            h = jnp.maximum(dist - _DELTA_V, 0.0)
            idx = lax.shift_left(gt_v, 4) + lanes
            plsc.addupdate_scatter(htab, [idx], h * h, mask=sel)

    vacc = zero
    for l in range(_NLAB):
        vacc = vacc + htab[pl.ds(l * _L, _L)] * w_vec[l]

    muf = [plsc.load_gather(mutab, [lab + _NLAB * f]) for f in range(_FEAT)]
    mu_halves = [mu01, mu01, mu23, mu23]
    pair_acc = zero
    for i in range(_NLAB):
        d2p = jnp.full((_L,), 1e-12, jnp.float32)
        for f in range(_FEAT):
            dfp = muf[f] - mu_halves[f][(f % 2) * _NLAB + i]
            d2p = d2p + dfp * dfp
        dp = _sqrt_nr(d2p)
        hp = jnp.maximum(2.0 * _DELTA_D - dp, 0.0)
        gate = jnp.where(lane_lt8 & (lab > i) & present, presf[i], 0.0)
        pair_acc = pair_acc + gate * hp * hp
    pair_denom = jnp.maximum(nf * (nf - 1.0), 1.0)
    dist_vec = (pair_acc * validf) / pair_denom

    r2 = jnp.full((_L,), 1e-12, jnp.float32)
    for f in range(_FEAT):
        r2 = r2 + muf[f] * muf[f]
    normv = _sqrt_nr(r2)
    reg_vec = (
        jnp.where(lane_lt8 & present, normv, 0.0) * (validf * (0.001 / _NB))
    ) / n_safe

    kf = (1 - jnp.minimum(k, 1)).astype(jnp.float32)
    obuf[...] = _LOSS_WEIGHT * (vacc + kf * (dist_vec + reg_vec))
    pltpu.sync_copy(obuf, out_hbm.at[c * 16 + s])


def kernel(emb, gt_instance, training_mask):
    out = _emb_loss_sc(emb, gt_instance, training_mask)
    return jnp.sum(out)


# speedup vs baseline: 1.0899x; 1.0899x over previous
import functools

import jax
import jax.numpy as jnp
from jax import lax
from jax.experimental import pallas as pl
from jax.experimental.pallas import tpu as pltpu
from jax.experimental.pallas import tpu_sc as plsc

_FEAT = 4
_NLAB = 8
_NB = 8
_H = 224
_W = 224
_ROWS = _H // 4
_L = 16
_CPR = _W // _L

_DELTA_V = 0.5
_DELTA_D = 1.5
_LOSS_WEIGHT = 0.25

_SUBS = ((0, 16), (16, 40))

_DMA_SEMS = len(_SUBS)


def _rsqrt_nr(x, iters=2):
    i = lax.bitcast_convert_type(x, jnp.int32)
    y = lax.bitcast_convert_type(
        jnp.int32(0x5F3759DF) - lax.shift_right_logical(i, 1), jnp.float32
    )
    for _ in range(iters):
        y = y * (1.5 - 0.5 * x * y * y)
    return y


def _sqrt_nr(x):
    return x * _rsqrt_nr(x, iters=3)


_mesh = plsc.VectorSubcoreMesh(
    core_axis_name="c", subcore_axis_name="s", num_cores=2, num_subcores=16
)


@functools.partial(
    pl.kernel,
    out_type=jax.ShapeDtypeStruct((32, _L), jnp.float32),
    mesh=_mesh,
    scratch_types=[
        pltpu.VMEM((_FEAT, _ROWS, _W), jnp.float32),
        pltpu.VMEM((_ROWS, _W), jnp.int32),
        pltpu.VMEM((_ROWS, _W), jnp.int32),
        pltpu.VMEM((640,), jnp.float32),
        pltpu.VMEM((128,), jnp.float32),
        pltpu.VMEM((640,), jnp.float32),
        pltpu.VMEM((2 * _L,), jnp.float32),
        pltpu.VMEM((_L,), jnp.float32),
        pltpu.VMEM_SHARED((16, 640), jnp.float32),
    ] + [pltpu.SemaphoreType.DMA] * _DMA_SEMS,
    compiler_params=pltpu.CompilerParams(needs_layout_passes=False),
)
def _emb_loss_sc(emb_hbm, gt_hbm, mask_hbm, out_hbm,
                 ev, gtv, mkv, acc, htab, tot, mutab, obuf, shacc, *sems):
    c = lax.axis_index("c")
    s = lax.axis_index("s")
    b = c * 4 + s // 4
    k = s % 4

    r0 = k * _ROWS
    copies = []
    for j, (rs, rn) in enumerate(_SUBS):
        cps = [
            pltpu.async_copy(gt_hbm.at[b, pl.ds(r0 + rs, rn), :],
                             gtv.at[pl.ds(rs, rn), :], sems[j]),
            pltpu.async_copy(mask_hbm.at[b, pl.ds(r0 + rs, rn), :],
                             mkv.at[pl.ds(rs, rn), :], sems[j]),
        ]
        for f in range(_FEAT):
            cps.append(pltpu.async_copy(
                emb_hbm.at[b, f, pl.ds(r0 + rs, rn), :],
                ev.at[f, pl.ds(rs, rn), :], sems[j]))
        copies.append(cps)

    lanes = lax.iota(jnp.int32, _L)
    zero = jnp.zeros((_L,), jnp.float32)
    ones = jnp.ones((_L,), jnp.float32)

    def _zero_acc(g, _):
        acc[pl.ds(g * _L, _L)] = zero
        return 0

    lax.fori_loop(0, 640 // _L, _zero_acc, 0)

    def _zero_htab(g, _):
        htab[pl.ds(g * _L, _L)] = zero
        return 0

    lax.fori_loop(0, 128 // _L, _zero_htab, 0)

    _HCOL = (_CPR // 2) * _L
    for j, (rs, rn) in enumerate(_SUBS):
        for cp in copies[j]:
            cp.wait()

        @plsc.parallel_loop(2 * rs, 2 * (rs + rn), step=1)
        def _p1(hr):
            r = lax.shift_right_logical(hr, 1)
            co = (hr & 1) * _HCOL
            for ci in range(_CPR // 2):
                cs = co + ci * _L
                gt_v = gtv[r, pl.ds(cs, _L)]
                mk = mkv[r, pl.ds(cs, _L)]
                sel = mk > 0
                idx = lax.shift_left(gt_v, 4) + lanes
                for f in range(_FEAT):
                    e = ev[f, r, pl.ds(cs, _L)]
                    plsc.addupdate_scatter(acc, [idx + f * 128], e, mask=sel)
                plsc.addupdate_scatter(acc, [idx + 4 * 128], ones, mask=sel)
                gtv[r, pl.ds(cs, _L)] = gt_v | lax.shift_left(mk, 3)

    pltpu.sync_copy(acc, shacc.at[s])
    plsc.subcore_barrier()
    row0 = (s // 4) * 4
    pltpu.sync_copy(shacc.at[row0], tot)

    def _merge(g, _):
        tot[pl.ds(g * _L, _L)] = tot[pl.ds(g * _L, _L)] + acc[pl.ds(g * _L, _L)]
        return 0

    for r in range(1, 4):
        pltpu.sync_copy(shacc.at[row0 + r], acc)
        lax.fori_loop(0, 640 // _L, _merge, 0)

    lab = lanes & 7
    half = lax.shift_right_logical(lanes, 3)
    base01 = lab * _L + half * 128

    def _lred(t, carry):
        a, b2, c2 = carry
        a = a + plsc.load_gather(tot, [base01 + t])
        b2 = b2 + plsc.load_gather(tot, [base01 + (256 + t)])
        c2 = c2 + plsc.load_gather(tot, [lab * _L + (512 + t)])
        return (a, b2, c2)

    s01, s23, scv = lax.fori_loop(0, _L, _lred, (zero, zero, zero))

    present = scv > 0.0
    presf = jnp.where(present, 1.0, 0.0)
    cntf = jnp.where(present, scv, 1.0)
    mu01 = s01 / cntf
    mu23 = s23 / cntf
    mutab[pl.ds(0, _L)] = mu01
    mutab[pl.ds(_L, _L)] = mu23

    lane_lt8 = lanes < 8
    nf = jnp.sum(jnp.where(lane_lt8, presf, 0.0))
    validf = jnp.minimum(jnp.maximum(nf - 1.0, 0.0), 1.0)
    n_safe = jnp.maximum(nf, 1.0)
    w_vec = (presf * validf) / (cntf * n_safe)

    @plsc.parallel_loop(0, 2 * _ROWS, step=1)
    def _p2(hr):
        r = lax.shift_right_logical(hr, 1)
        co = (hr & 1) * _HCOL
        for ci in range(_CPR // 2):
            cs = co + ci * _L
            gm = gtv[r, pl.ds(cs, _L)]
            gt_v = gm & 7
            sel = gm > 7
            d2 = jnp.full((_L,), 1e-12, jnp.float32)
            for f in range(_FEAT):
                e = ev[f, r, pl.ds(cs, _L)]
                mu = plsc.load_gather(mutab, [gt_v + f * _NLAB])
                df = e - mu
                d2 = d2 + df * df
            dist = d2 * _rsqrt_nr(d2, iters=1)
            h = jnp.maximum(dist - _DELTA_V, 0.0)
            idx = lax.shift_left(gt_v, 4) + lanes
            plsc.addupdate_scatter(htab, [idx], h * h, mask=sel)

    vacc = zero
    for l in range(_NLAB):
        vacc = vacc + htab[pl.ds(l * _L, _L)] * w_vec[l]

    muf = [plsc.load_gather(mutab, [lab + _NLAB * f]) for f in range(_FEAT)]
    mu_halves = [mu01, mu01, mu23, mu23]
    pair_acc = zero
    for i in range(_NLAB):
        d2p = jnp.full((_L,), 1e-12, jnp.float32)
        for f in range(_FEAT):
            dfp = muf[f] - mu_halves[f][(f % 2) * _NLAB + i]
            d2p = d2p + dfp * dfp
        dp = _sqrt_nr(d2p)
        hp = jnp.maximum(2.0 * _DELTA_D - dp, 0.0)
        gate = jnp.where(lane_lt8 & (lab > i) & present, presf[i], 0.0)
        pair_acc = pair_acc + gate * hp * hp
    pair_denom = jnp.maximum(nf * (nf - 1.0), 1.0)
    dist_vec = (pair_acc * validf) / pair_denom

    r2 = jnp.full((_L,), 1e-12, jnp.float32)
    for f in range(_FEAT):
        r2 = r2 + muf[f] * muf[f]
    normv = _sqrt_nr(r2)
    reg_vec = (
        jnp.where(lane_lt8 & present, normv, 0.0) * (validf * (0.001 / _NB))
    ) / n_safe

    kf = (1 - jnp.minimum(k, 1)).astype(jnp.float32)
    obuf[...] = _LOSS_WEIGHT * (vacc + kf * (dist_vec + reg_vec))
    pltpu.sync_copy(obuf, out_hbm.at[c * 16 + s])


def kernel(emb, gt_instance, training_mask):
    out = _emb_loss_sc(emb, gt_instance, training_mask)
    return jnp.sum(out)
